# Initial kernel scaffold; baseline (speedup 1.0000x reference)
import jax
import jax.numpy as jnp
from jax.experimental import pallas as pl


def kernel(x, edge_index, batch, w1_0, b1_0, w2_0, b2_0, gamma_0, beta_0, w1_1, b1_1, w2_1, b2_1, gamma_1, beta_1, w1_2, b1_2, w2_2, b2_2, gamma_2, beta_2):
    raise NotImplementedError("write your pallas kernel here")


# speedup vs baseline: 6.1658x; 6.1658x over previous
import functools

import jax
import jax.numpy as jnp
from jax import lax
from jax.experimental import pallas as pl
from jax.experimental.pallas import tpu as pltpu
from jax.experimental.pallas import tpu_sc as plsc

_N = 10000
_E = 320000
_D = 128
_H = 128
_G = 128

_NC = 2
_NS = 16
_NW = _NC * _NS
_EPW = _E // _NW
_CHUNK = 80
_NCHUNK = _EPW // _CHUNK
_WB_BASE = 624
_WB_ROWS = 640


def _sc_agg_body(z_hbm, src_hbm, dst_hbm, zero_hbm, out_hbm,
                 src_idx, dst_idx, rows_v, agg_sh, sem):
    c = lax.axis_index("c")
    s = lax.axis_index("s")
    wid = c * _NS + s

    pltpu.sync_copy(zero_hbm, agg_sh.at[pl.ds(s * _WB_BASE, _WB_ROWS)])

    pltpu.sync_copy(src_hbm.at[wid], src_idx)
    pltpu.sync_copy(dst_hbm.at[wid], dst_idx)

    plsc.subcore_barrier()

    def body(k, carry):
        pltpu.async_copy(z_hbm.at[src_idx.at[k]], rows_v, sem).wait()
        pltpu.sync_copy(rows_v, agg_sh.at[dst_idx.at[k]], add=True)
        return carry

    lax.fori_loop(0, _NCHUNK, body, 0)

    plsc.subcore_barrier()

    pltpu.sync_copy(agg_sh.at[pl.ds(s * _WB_BASE, _WB_ROWS)],
                    out_hbm.at[c, pl.ds(s * _WB_BASE, _WB_ROWS)])


@jax.jit
def _sc_agg(z, src_r, dst_r, zeros_tile):
    mesh = plsc.VectorSubcoreMesh(core_axis_name="c", subcore_axis_name="s")
    k = pl.kernel(
        _sc_agg_body,
        out_type=jax.ShapeDtypeStruct((_NC, _N, _D), jnp.float32),
        mesh=mesh,
        scratch_types=[
            pltpu.VMEM((_NCHUNK, _CHUNK), jnp.int32),
            pltpu.VMEM((_NCHUNK, _CHUNK), jnp.int32),
            pltpu.VMEM((_CHUNK, _D), jnp.float32),
            pltpu.VMEM_SHARED((_N, _D), jnp.float32),
            pltpu.SemaphoreType.DMA,
        ],
    )
    return k(z, src_r, dst_r, zeros_tile)


_BLK = 1000


def _mlp_body(z_ref, a0_ref, a1_ref, w1_ref, b1_ref, w2_ref, b2_ref,
              h_ref, st_ref):
    h = z_ref[...] + a0_ref[...] + a1_ref[...]
    u = jnp.dot(h, w1_ref[...], preferred_element_type=jnp.float32)
    u = jnp.maximum(u + b1_ref[...], 0.0)
    v = jnp.dot(u, w2_ref[...], preferred_element_type=jnp.float32)
    v = jnp.maximum(v + b2_ref[...], 0.0)
    h_ref[...] = v

    @pl.when(pl.program_id(0) == 0)
    def _():
        st_ref[...] = jnp.zeros_like(st_ref)

    inc = jnp.concatenate(
        [jnp.sum(v, axis=0)[None, :], jnp.sum(v * v, axis=0)[None, :]], axis=0)
    st_ref[...] += inc


@jax.jit
def _tc_mlp(z, a0, a1, w1, b1, w2, b2):
    grid = (_N // _BLK,)
    return pl.pallas_call(
        _mlp_body,
        grid=grid,
        in_specs=[
            pl.BlockSpec((_BLK, _D), lambda i: (i, 0)),
            pl.BlockSpec((_BLK, _D), lambda i: (i, 0)),
            pl.BlockSpec((_BLK, _D), lambda i: (i, 0)),
            pl.BlockSpec((_D, _H), lambda i: (0, 0)),
            pl.BlockSpec((1, _H), lambda i: (0, 0)),
            pl.BlockSpec((_H, _H), lambda i: (0, 0)),
            pl.BlockSpec((1, _H), lambda i: (0, 0)),
        ],
        out_specs=[
            pl.BlockSpec((_BLK, _H), lambda i: (i, 0)),
            pl.BlockSpec((2, _H), lambda i: (0, 0)),
        ],
        out_shape=[
            jax.ShapeDtypeStruct((_N, _H), jnp.float32),
            jax.ShapeDtypeStruct((2, _H), jnp.float32),
        ],
    )(z, a0, a1, w1, b1, w2, b2)


def _norm_body(h_ref, st_ref, gm_ref, bt_ref, bidx_ref, o_ref, p_ref):
    inv_n = 1.0 / _N
    mean = st_ref[0, :] * inv_n
    var = st_ref[1, :] * inv_n - mean * mean
    a = gm_ref[0, :] * lax.rsqrt(var + 1e-5)
    cc = bt_ref[0, :] - mean * a
    hn = h_ref[...] * a[None, :] + cc[None, :]
    o_ref[...] = hn

    b = bidx_ref[0, :]
    oh = (lax.broadcasted_iota(jnp.int32, (_G, _BLK), 0) == b[None, :])
    oh = oh.astype(jnp.float32)

    @pl.when(pl.program_id(0) == 0)
    def _():
        p_ref[...] = jnp.zeros_like(p_ref)

    p_ref[...] += jnp.dot(oh, hn, preferred_element_type=jnp.float32)


@jax.jit
def _tc_norm(h, st, gamma, beta, batch3):
    grid = (_N // _BLK,)
    return pl.pallas_call(
        _norm_body,
        grid=grid,
        in_specs=[
            pl.BlockSpec((_BLK, _H), lambda i: (i, 0)),
            pl.BlockSpec((2, _H), lambda i: (0, 0)),
            pl.BlockSpec((1, _H), lambda i: (0, 0)),
            pl.BlockSpec((1, _H), lambda i: (0, 0)),
            pl.BlockSpec((None, 1, _BLK), lambda i: (i, 0, 0)),
        ],
        out_specs=[
            pl.BlockSpec((_BLK, _H), lambda i: (i, 0)),
            pl.BlockSpec((_G, _H), lambda i: (0, 0)),
        ],
        out_shape=[
            jax.ShapeDtypeStruct((_N, _H), jnp.float32),
            jax.ShapeDtypeStruct((_G, _H), jnp.float32),
        ],
    )(h, st, gamma, beta, batch3)


def kernel(x, edge_index, batch,
           w1_0, b1_0, w2_0, b2_0, gamma_0, beta_0,
           w1_1, b1_1, w2_1, b2_1, gamma_1, beta_1,
           w1_2, b1_2, w2_2, b2_2, gamma_2, beta_2):
    params = [
        (w1_0, b1_0, w2_0, b2_0, gamma_0, beta_0),
        (w1_1, b1_1, w2_1, b2_1, gamma_1, beta_1),
        (w1_2, b1_2, w2_2, b2_2, gamma_2, beta_2),
    ]
    src_r = edge_index[0].reshape(_NW, _NCHUNK, _CHUNK)
    dst_r = edge_index[1].reshape(_NW, _NCHUNK, _CHUNK)
    zeros_tile = jnp.zeros((_WB_ROWS, _D), jnp.float32)
    batch3 = batch.reshape(_N // _BLK, 1, _BLK)

    z = x
    zs = []
    gs = []
    for (w1, b1, w2, b2, gamma, beta) in params:
        agg = _sc_agg(z, src_r, dst_r, zeros_tile)
        h, st = _tc_mlp(z, agg[0], agg[1], w1, b1.reshape(1, _H),
                        w2, b2.reshape(1, _H))
        hn, pooled = _tc_norm(h, st, gamma.reshape(1, _H),
                              beta.reshape(1, _H), batch3)
        zs.append(hn)
        gs.append(pooled)
        z = hn

    z_out = jnp.concatenate(zs, axis=1)
    g_out = jnp.concatenate(gs, axis=1)
    return z_out, g_out


# speedup vs baseline: 11.1161x; 1.8029x over previous
import functools

import jax
import jax.numpy as jnp
from jax import lax
from jax.experimental import pallas as pl
from jax.experimental.pallas import tpu as pltpu
from jax.experimental.pallas import tpu_sc as plsc

_N = 10000
_E = 320000
_D = 128
_H = 128
_G = 128

_NC = 2
_NS = 16
_NW = _NC * _NS
_EPW = _E // _NW
_CHUNK = 80
_NCHUNK = _EPW // _CHUNK
_WB_BASE = 624
_WB_ROWS = 640


_NBUF = 3
_NITER = _NCHUNK // _NBUF
_NREM = _NCHUNK - _NITER * _NBUF


def _sc_agg_body(z_hbm, src_hbm, dst_hbm, zero_hbm, out_hbm,
                 src_idx, dst_ring, rows_v, agg_sh, gsems, ssems):

    def fire(k, b):
        pltpu.async_copy(z_hbm.at[src_idx.at[pl.ds(k * _CHUNK, _CHUNK)]],
                         rows_v.at[b], gsems.at[b])
        pltpu.async_copy(dst_hbm.at[wid, k], dst_ring.at[b], gsems.at[b])

    def drain_gather(b):
        pltpu.make_async_copy(z_hbm.at[src_idx.at[pl.ds(0, _CHUNK)]],
                              rows_v.at[b], gsems.at[b]).wait()
        pltpu.make_async_copy(dst_hbm.at[wid, 0], dst_ring.at[b],
                              gsems.at[b]).wait()

    def fire_scatter(b):
        pltpu.async_copy(rows_v.at[b], agg_sh.at[dst_ring.at[b]], ssems.at[b],
                         add=True)

    def drain_scatter(b):
        pltpu.make_async_copy(rows_v.at[b], agg_sh.at[dst_ring.at[b]],
                              ssems.at[b]).wait()

    c = lax.axis_index("c")
    s = lax.axis_index("s")
    wid = c * _NS + s

    pltpu.sync_copy(src_hbm.at[wid], src_idx)

    pltpu.sync_copy(zero_hbm, agg_sh.at[pl.ds(s * _WB_BASE, _WB_ROWS)])

    for b in range(_NBUF):
        fire(b, b)

    plsc.subcore_barrier()

    def body(t, carry):
        base = t * _NBUF
        for b in range(_NBUF):
            k = base + b
            drain_gather(b)
            fire_scatter(b)
            drain_scatter(b)
            nxt = k + _NBUF

            @pl.when(nxt < _NCHUNK)
            def _():
                fire(nxt, b)
        return carry

    lax.fori_loop(0, _NITER, body, 0)

    for b in range(_NREM):
        drain_gather(b)
        fire_scatter(b)
        drain_scatter(b)

    plsc.subcore_barrier()

    pltpu.sync_copy(agg_sh.at[pl.ds(s * _WB_BASE, _WB_ROWS)],
                    out_hbm.at[c, pl.ds(s * _WB_BASE, _WB_ROWS)])


@jax.jit
def _sc_agg(z, src_r, dst_r, zeros_tile):
    mesh = plsc.VectorSubcoreMesh(core_axis_name="c", subcore_axis_name="s")
    k = pl.kernel(
        _sc_agg_body,
        out_type=jax.ShapeDtypeStruct((_NC, _N, _D), jnp.float32),
        mesh=mesh,
        scratch_types=[
            pltpu.VMEM((_EPW,), jnp.int32),
            pltpu.VMEM((_NBUF, _CHUNK), jnp.int32),
            pltpu.VMEM((_NBUF, _CHUNK, _D), jnp.float32),
            pltpu.VMEM_SHARED((_N, _D), jnp.float32),
            pltpu.SemaphoreType.DMA((_NBUF,)),
            pltpu.SemaphoreType.DMA((_NBUF,)),
        ],
    )
    return k(z, src_r, dst_r, zeros_tile)


_BLK = 1000


def _mlp_body(z_ref, a0_ref, a1_ref, w1_ref, b1_ref, w2_ref, b2_ref,
              h_ref, st_ref):
    h = z_ref[...] + a0_ref[...] + a1_ref[...]
    u = jnp.dot(h, w1_ref[...], preferred_element_type=jnp.float32)
    u = jnp.maximum(u + b1_ref[...], 0.0)
    v = jnp.dot(u, w2_ref[...], preferred_element_type=jnp.float32)
    v = jnp.maximum(v + b2_ref[...], 0.0)
    h_ref[...] = v

    @pl.when(pl.program_id(0) == 0)
    def _():
        st_ref[...] = jnp.zeros_like(st_ref)

    inc = jnp.concatenate(
        [jnp.sum(v, axis=0)[None, :], jnp.sum(v * v, axis=0)[None, :]], axis=0)
    st_ref[...] += inc


@jax.jit
def _tc_mlp(z, a0, a1, w1, b1, w2, b2):
    grid = (_N // _BLK,)
    return pl.pallas_call(
        _mlp_body,
        grid=grid,
        in_specs=[
            pl.BlockSpec((_BLK, _D), lambda i: (i, 0)),
            pl.BlockSpec((_BLK, _D), lambda i: (i, 0)),
            pl.BlockSpec((_BLK, _D), lambda i: (i, 0)),
            pl.BlockSpec((_D, _H), lambda i: (0, 0)),
            pl.BlockSpec((1, _H), lambda i: (0, 0)),
            pl.BlockSpec((_H, _H), lambda i: (0, 0)),
            pl.BlockSpec((1, _H), lambda i: (0, 0)),
        ],
        out_specs=[
            pl.BlockSpec((_BLK, _H), lambda i: (i, 0)),
            pl.BlockSpec((2, _H), lambda i: (0, 0)),
        ],
        out_shape=[
            jax.ShapeDtypeStruct((_N, _H), jnp.float32),
            jax.ShapeDtypeStruct((2, _H), jnp.float32),
        ],
    )(z, a0, a1, w1, b1, w2, b2)


def _norm_body(h_ref, st_ref, gm_ref, bt_ref, bidx_ref, o_ref, p_ref):
    inv_n = 1.0 / _N
    mean = st_ref[0, :] * inv_n
    var = st_ref[1, :] * inv_n - mean * mean
    a = gm_ref[0, :] * lax.rsqrt(var + 1e-5)
    cc = bt_ref[0, :] - mean * a
    hn = h_ref[...] * a[None, :] + cc[None, :]
    o_ref[...] = hn

    b = bidx_ref[0, :]
    oh = (lax.broadcasted_iota(jnp.int32, (_G, _BLK), 0) == b[None, :])
    oh = oh.astype(jnp.float32)

    @pl.when(pl.program_id(0) == 0)
    def _():
        p_ref[...] = jnp.zeros_like(p_ref)

    p_ref[...] += jnp.dot(oh, hn, preferred_element_type=jnp.float32)


@jax.jit
def _tc_norm(h, st, gamma, beta, batch3):
    grid = (_N // _BLK,)
    return pl.pallas_call(
        _norm_body,
        grid=grid,
        in_specs=[
            pl.BlockSpec((_BLK, _H), lambda i: (i, 0)),
            pl.BlockSpec((2, _H), lambda i: (0, 0)),
            pl.BlockSpec((1, _H), lambda i: (0, 0)),
            pl.BlockSpec((1, _H), lambda i: (0, 0)),
            pl.BlockSpec((None, 1, _BLK), lambda i: (i, 0, 0)),
        ],
        out_specs=[
            pl.BlockSpec((_BLK, _H), lambda i: (i, 0)),
            pl.BlockSpec((_G, _H), lambda i: (0, 0)),
        ],
        out_shape=[
            jax.ShapeDtypeStruct((_N, _H), jnp.float32),
            jax.ShapeDtypeStruct((_G, _H), jnp.float32),
        ],
    )(h, st, gamma, beta, batch3)


def kernel(x, edge_index, batch,
           w1_0, b1_0, w2_0, b2_0, gamma_0, beta_0,
           w1_1, b1_1, w2_1, b2_1, gamma_1, beta_1,
           w1_2, b1_2, w2_2, b2_2, gamma_2, beta_2):
    params = [
        (w1_0, b1_0, w2_0, b2_0, gamma_0, beta_0),
        (w1_1, b1_1, w2_1, b2_1, gamma_1, beta_1),
        (w1_2, b1_2, w2_2, b2_2, gamma_2, beta_2),
    ]
    src_r = edge_index[0].reshape(_NW, _EPW)
    dst_r = edge_index[1].reshape(_NW, _NCHUNK, _CHUNK)
    zeros_tile = jnp.zeros((_WB_ROWS, _D), jnp.float32)
    batch3 = batch.reshape(_N // _BLK, 1, _BLK)

    z = x
    zs = []
    gs = []
    for (w1, b1, w2, b2, gamma, beta) in params:
        agg = _sc_agg(z, src_r, dst_r, zeros_tile)
        h, st = _tc_mlp(z, agg[0], agg[1], w1, b1.reshape(1, _H),
                        w2, b2.reshape(1, _H))
        hn, pooled = _tc_norm(h, st, gamma.reshape(1, _H),
                              beta.reshape(1, _H), batch3)
        zs.append(hn)
        gs.append(pooled)
        z = hn

    z_out = jnp.concatenate(zs, axis=1)
    g_out = jnp.concatenate(gs, axis=1)
    return z_out, g_out


# speedup vs baseline: 11.8563x; 1.0666x over previous
import functools

import jax
import jax.numpy as jnp
from jax import lax
from jax.experimental import pallas as pl
from jax.experimental.pallas import tpu as pltpu
from jax.experimental.pallas import tpu_sc as plsc

_N = 10000
_E = 320000
_D = 128
_H = 128
_G = 128

_NC = 2
_NS = 16
_NW = _NC * _NS
_EPW = _E // _NW
_CHUNK = 80
_NCHUNK = _EPW // _CHUNK
_WB_BASE = 624
_WB_ROWS = 640


_NBUF = 3
_NITER = _NCHUNK // _NBUF
_NREM = _NCHUNK - _NITER * _NBUF


def _sc_agg_body(z_hbm, src_hbm, dst_hbm, zero_hbm, out_hbm,
                 src_idx, dst_ring, rows_v, agg_sh, gsems, ssems):

    def fire(k, b):
        pltpu.async_copy(z_hbm.at[src_idx.at[pl.ds(k * _CHUNK, _CHUNK)]],
                         rows_v.at[b], gsems.at[b])
        pltpu.async_copy(dst_hbm.at[wid, k], dst_ring.at[b], gsems.at[b])

    def drain_gather(b):
        pltpu.make_async_copy(z_hbm.at[src_idx.at[pl.ds(0, _CHUNK)]],
                              rows_v.at[b], gsems.at[b]).wait()
        pltpu.make_async_copy(dst_hbm.at[wid, 0], dst_ring.at[b],
                              gsems.at[b]).wait()

    def fire_scatter(b):
        pltpu.async_copy(rows_v.at[b], agg_sh.at[dst_ring.at[b]], ssems.at[b],
                         add=True)

    def drain_scatter(b):
        pltpu.make_async_copy(rows_v.at[b], agg_sh.at[dst_ring.at[b]],
                              ssems.at[b]).wait()

    c = lax.axis_index("c")
    s = lax.axis_index("s")
    wid = c * _NS + s

    pltpu.sync_copy(src_hbm.at[wid], src_idx)

    pltpu.sync_copy(zero_hbm, agg_sh.at[pl.ds(s * _WB_BASE, _WB_ROWS)])

    for b in range(_NBUF):
        fire(b, b)

    plsc.subcore_barrier()

    def body(t, carry):
        base = t * _NBUF
        for b in range(_NBUF):
            k = base + b
            drain_gather(b)
            fire_scatter(b)
            drain_scatter(b)
            nxt = k + _NBUF

            @pl.when(nxt < _NCHUNK)
            def _():
                fire(nxt, b)
        return carry

    lax.fori_loop(0, _NITER, body, 0)

    for b in range(_NREM):
        drain_gather(b)
        fire_scatter(b)
        drain_scatter(b)

    plsc.subcore_barrier()

    pltpu.sync_copy(agg_sh.at[pl.ds(s * _WB_BASE, _WB_ROWS)],
                    out_hbm.at[c, pl.ds(s * _WB_BASE, _WB_ROWS)])


@jax.jit
def _sc_agg(z, src_r, dst_r, zeros_tile):
    mesh = plsc.VectorSubcoreMesh(core_axis_name="c", subcore_axis_name="s")
    k = pl.kernel(
        _sc_agg_body,
        out_type=jax.ShapeDtypeStruct((_NC, _N, _D), jnp.float32),
        mesh=mesh,
        scratch_types=[
            pltpu.VMEM((_EPW,), jnp.int32),
            pltpu.VMEM((_NBUF, _CHUNK), jnp.int32),
            pltpu.VMEM((_NBUF, _CHUNK, _D), jnp.float32),
            pltpu.VMEM_SHARED((_N, _D), jnp.float32),
            pltpu.SemaphoreType.DMA((_NBUF,)),
            pltpu.SemaphoreType.DMA((_NBUF,)),
        ],
    )
    return k(z, src_r, dst_r, zeros_tile)


_BLK = 2000
_NBLK = _N // _BLK


def _layer_body(z_ref, a0_ref, a1_ref, w1_ref, b1_ref, w2_ref, b2_ref,
                gm_ref, bt_ref, bidx_ref, o_ref, p_ref, h_scr, st_scr):
    p = pl.program_id(0)
    i = pl.program_id(1)

    @pl.when(p == 0)
    def _():
        h = z_ref[...] + a0_ref[...] + a1_ref[...]
        u = jnp.dot(h, w1_ref[...], preferred_element_type=jnp.float32)
        u = jnp.maximum(u + b1_ref[...], 0.0)
        v = jnp.dot(u, w2_ref[...], preferred_element_type=jnp.float32)
        v = jnp.maximum(v + b2_ref[...], 0.0)
        h_scr[pl.ds(i * _BLK, _BLK), :] = v

        @pl.when(i == 0)
        def _():
            st_scr[...] = jnp.zeros_like(st_scr)

        inc = jnp.concatenate(
            [jnp.sum(v, axis=0)[None, :], jnp.sum(v * v, axis=0)[None, :]],
            axis=0)
        st_scr[0:2, :] += inc

    @pl.when(p == 1)
    def _():
        inv_n = 1.0 / _N
        mean = st_scr[0, :] * inv_n
        var = st_scr[1, :] * inv_n - mean * mean
        a = gm_ref[0, :] * lax.rsqrt(var + 1e-5)
        cc = bt_ref[0, :] - mean * a
        hp = h_scr[pl.ds(i * _BLK, _BLK), :]
        hn = hp * a[None, :] + cc[None, :]
        o_ref[...] = hn

        b = bidx_ref[0, :]
        oh = (lax.broadcasted_iota(jnp.int32, (_G, _BLK), 0) == b[None, :])
        oh = oh.astype(jnp.float32)

        @pl.when(i == 0)
        def _():
            p_ref[...] = jnp.zeros_like(p_ref)

        p_ref[...] += jnp.dot(oh, hn, preferred_element_type=jnp.float32)


@jax.jit
def _tc_layer(z, a0, a1, w1, b1, w2, b2, gamma, beta, batch3):
    return pl.pallas_call(
        _layer_body,
        grid=(2, _NBLK),
        in_specs=[
            pl.BlockSpec((_BLK, _D), lambda p, i: (i * (1 - p), 0)),
            pl.BlockSpec((_BLK, _D), lambda p, i: (i * (1 - p), 0)),
            pl.BlockSpec((_BLK, _D), lambda p, i: (i * (1 - p), 0)),
            pl.BlockSpec((_D, _H), lambda p, i: (0, 0)),
            pl.BlockSpec((1, _H), lambda p, i: (0, 0)),
            pl.BlockSpec((_H, _H), lambda p, i: (0, 0)),
            pl.BlockSpec((1, _H), lambda p, i: (0, 0)),
            pl.BlockSpec((1, _H), lambda p, i: (0, 0)),
            pl.BlockSpec((1, _H), lambda p, i: (0, 0)),
            pl.BlockSpec((None, 1, _BLK), lambda p, i: (i, 0, 0)),
        ],
        out_specs=[
            pl.BlockSpec((_BLK, _H), lambda p, i: (i * p, 0)),
            pl.BlockSpec((_G, _H), lambda p, i: (0, 0)),
        ],
        out_shape=[
            jax.ShapeDtypeStruct((_N, _H), jnp.float32),
            jax.ShapeDtypeStruct((_G, _H), jnp.float32),
        ],
        scratch_shapes=[
            pltpu.VMEM((_N, _H), jnp.float32),
            pltpu.VMEM((8, _H), jnp.float32),
        ],
    )(z, a0, a1, w1, b1, w2, b2, gamma, beta, batch3)


def kernel(x, edge_index, batch,
           w1_0, b1_0, w2_0, b2_0, gamma_0, beta_0,
           w1_1, b1_1, w2_1, b2_1, gamma_1, beta_1,
           w1_2, b1_2, w2_2, b2_2, gamma_2, beta_2):
    params = [
        (w1_0, b1_0, w2_0, b2_0, gamma_0, beta_0),
        (w1_1, b1_1, w2_1, b2_1, gamma_1, beta_1),
        (w1_2, b1_2, w2_2, b2_2, gamma_2, beta_2),
    ]
    src_r = edge_index[0].reshape(_NW, _EPW)
    dst_r = edge_index[1].reshape(_NW, _NCHUNK, _CHUNK)
    zeros_tile = jnp.zeros((_WB_ROWS, _D), jnp.float32)
    batch3 = batch.reshape(_NBLK, 1, _BLK)

    z = x
    zs = []
    gs = []
    for (w1, b1, w2, b2, gamma, beta) in params:
        agg = _sc_agg(z, src_r, dst_r, zeros_tile)
        hn, pooled = _tc_layer(z, agg[0], agg[1], w1, b1.reshape(1, _H),
                               w2, b2.reshape(1, _H), gamma.reshape(1, _H),
                               beta.reshape(1, _H), batch3)
        zs.append(hn)
        gs.append(pooled)
        z = hn

    z_out = jnp.concatenate(zs, axis=1)
    g_out = jnp.concatenate(gs, axis=1)
    return z_out, g_out
